# baseline (speedup 1.0000x reference)
import functools

import jax
import jax.numpy as jnp
from jax import lax
from jax.experimental import pallas as pl
from jax.experimental.pallas import tpu as pltpu
from jax.experimental.pallas import tpu_sc as plsc

B = 1024
NUM = 13
NCAT = 26
V = 1000
D = 16
NCOLS = 64
ROW_BLK = 128


def _stage1_body(x_ref, m_ref, idx_ref, sparse_ref):
    x = x_ref[...]
    m = m_ref[...]
    o = lax.dot_general(x, m, (((1,), (0,)), ((), ())),
                        preferred_element_type=jnp.float32)
    lab = o[:, :NCAT] * 128.0 + o[:, NCAT:2 * NCAT]
    lab_i = (lab + 0.5).astype(jnp.int32)
    offs = lax.broadcasted_iota(jnp.int32, (1, NCAT), 1) * V
    idx_ref[...] = lab_i + offs
    sparse_ref[...] = o[:, 2 * NCAT:2 * NCAT + 1]


def _stage1(x, m):
    return pl.pallas_call(
        _stage1_body,
        grid=(B // ROW_BLK,),
        in_specs=[
            pl.BlockSpec((ROW_BLK, NCAT * V), lambda i: (i, 0)),
            pl.BlockSpec((NCAT * V, NCOLS), lambda i: (0, 0)),
        ],
        out_specs=[
            pl.BlockSpec((ROW_BLK, NCAT), lambda i: (i, 0)),
            pl.BlockSpec((ROW_BLK, 1), lambda i: (i, 0)),
        ],
        out_shape=[
            jax.ShapeDtypeStruct((B, NCAT), jnp.int32),
            jax.ShapeDtypeStruct((B, 1), jnp.float32),
        ],
    )(x, m)


def _sc_gather(table, idx):
    info = plsc.get_sparse_core_info()
    nw = info.num_cores * info.num_subcores
    n = idx.shape[0]
    b_per_w = n // nw
    mesh = plsc.VectorSubcoreMesh(core_axis_name="c", subcore_axis_name="s")

    @functools.partial(
        pl.kernel, mesh=mesh,
        compiler_params=pltpu.CompilerParams(use_tc_tiling_on_sc=False),
        out_type=jax.ShapeDtypeStruct((n, D), jnp.float32),
        scratch_types=[
            pltpu.VMEM((b_per_w,), jnp.int32),
            pltpu.VMEM((b_per_w, D), jnp.float32),
            pltpu.SemaphoreType.DMA,
        ],
    )
    def k(table_hbm, idx_hbm, out_hbm, idx_v, rows_v, sem):
        wid = lax.axis_index("s") * info.num_cores + lax.axis_index("c")
        base = wid * b_per_w
        pltpu.sync_copy(idx_hbm.at[pl.ds(base, b_per_w)], idx_v)
        pltpu.async_copy(table_hbm.at[idx_v], rows_v, sem).wait()
        pltpu.sync_copy(rows_v, out_hbm.at[pl.ds(base, b_per_w)])

    return k(table, idx)


def _stage3_body(num_ref, cat_ref, sp_ref, wnum_ref, ncst_ref, s_ref, a_ref,
                 w1c_ref, b1_ref, w2_ref, b2_ref, w3_ref, b3_ref, wfm_ref,
                 bfm_ref, out_ref):
    hp = lax.Precision.HIGHEST
    num = num_ref[...]
    cat = cat_ref[...]
    nsum = jnp.dot(num, wnum_ref[...], precision=hp) + ncst_ref[...]
    csum = jnp.dot(cat, s_ref[...], precision=hp)
    yfme = jnp.sum(nsum * csum, axis=1, keepdims=True)
    yfms = jnp.dot(num, wfm_ref[...], precision=hp) + sp_ref[...] + bfm_ref[...]
    h1 = jnp.maximum(
        jnp.dot(num, a_ref[...], precision=hp)
        + jnp.dot(cat, w1c_ref[...], precision=hp) + b1_ref[...], 0.0)
    h2 = jnp.maximum(jnp.dot(h1, w2_ref[...], precision=hp) + b2_ref[...], 0.0)
    yd = jnp.dot(h2, w3_ref[...], precision=hp) + b3_ref[...]
    out_ref[...] = jax.nn.sigmoid(yfme + yfms + yd)


def _stage3(num, catf, sparse, wnum, ncst, s, a, w1c, b1c, w2, b2, w3, b3,
            wfm13, bfm):
    return pl.pallas_call(
        _stage3_body,
        out_shape=jax.ShapeDtypeStruct((B, 1), jnp.float32),
    )(num, catf, sparse, wnum, ncst, s, a, w1c, b1c, w2, b2, w3, b3,
      wfm13, bfm)


def kernel(numeric_feats, categorical_feats, W_num, b_num, emb_tables,
           W_fm, b_fm, W1, b1, W2, b2, W3, b3):
    f32 = jnp.float32
    k = jnp.arange(NCAT * V)
    field = k // V
    r = k % V
    hi = (r // 128).astype(f32)
    lo = (r % 128).astype(f32)
    m = (jax.nn.one_hot(field, NCOLS, dtype=f32) * hi[:, None]
         + jax.nn.one_hot(field + NCAT, NCOLS, dtype=f32) * lo[:, None])
    m = m.at[:, 2 * NCAT].set(W_fm[NUM:, 0])

    a = jnp.einsum('id,idm->im', W_num, W1[:NUM * D].reshape(NUM, D, -1))
    b1c = (b1 + b_num.reshape(-1) @ W1[:NUM * D])[None]
    ncst = b_num.sum(0)[None]
    s = jnp.tile(jnp.eye(D, dtype=f32), (NCAT, 1))

    idx2d, sparse = _stage1(categorical_feats, m)

    cat_rows = _sc_gather(emb_tables.reshape(NCAT * V, D), idx2d.reshape(-1))
    catf = cat_rows.reshape(B, NCAT * D)

    return _stage3(numeric_feats, catf, sparse, W_num, ncst, s, a,
                   W1[NUM * D:], b1c, W2, b2[None], W3, b3[None],
                   W_fm[:NUM], b_fm[None])


# speedup vs baseline: 1.3496x; 1.3496x over previous
import functools

import jax
import jax.numpy as jnp
from jax import lax
from jax.experimental import pallas as pl
from jax.experimental.pallas import tpu as pltpu
from jax.experimental.pallas import tpu_sc as plsc

B = 1024
NUM = 13
NCAT = 26
V = 1000
D = 16
NCOLS = 64
ROW_BLK = 128


def _stage1_body(x_ref, m_ref, idx_ref, sparse_ref):
    x = x_ref[...]
    m = m_ref[...]
    o = lax.dot_general(x, m, (((1,), (0,)), ((), ())),
                        preferred_element_type=jnp.float32)
    lab = o[:, :NCAT] * 128.0 + o[:, NCAT:2 * NCAT]
    lab_i = (lab + 0.5).astype(jnp.int32)
    offs = lax.broadcasted_iota(jnp.int32, (1, NCAT), 1) * V
    idx_ref[...] = lab_i + offs
    sparse_ref[...] = o[:, 2 * NCAT:2 * NCAT + 1]


def _stage1(x, m):
    return pl.pallas_call(
        _stage1_body,
        grid=(B // ROW_BLK,),
        in_specs=[
            pl.BlockSpec((ROW_BLK, NCAT * V), lambda i: (i, 0)),
            pl.BlockSpec((NCAT * V, NCOLS), lambda i: (0, 0)),
        ],
        out_specs=[
            pl.BlockSpec((ROW_BLK, NCAT), lambda i: (i, 0)),
            pl.BlockSpec((ROW_BLK, 1), lambda i: (i, 0)),
        ],
        out_shape=[
            jax.ShapeDtypeStruct((B, NCAT), jnp.int32),
            jax.ShapeDtypeStruct((B, 1), jnp.float32),
        ],
    )(x, m)


def _sc_gather(table, idx):
    info = plsc.get_sparse_core_info()
    nw = info.num_cores * info.num_subcores
    n = idx.shape[0]
    b_per_w = n // nw
    mesh = plsc.VectorSubcoreMesh(core_axis_name="c", subcore_axis_name="s")

    @functools.partial(
        pl.kernel, mesh=mesh,
        compiler_params=pltpu.CompilerParams(use_tc_tiling_on_sc=False),
        out_type=jax.ShapeDtypeStruct((n, D), jnp.float32),
        scratch_types=[
            pltpu.VMEM((b_per_w,), jnp.int32),
            pltpu.VMEM((b_per_w, D), jnp.float32),
            pltpu.SemaphoreType.DMA,
        ],
    )
    def k(table_hbm, idx_hbm, out_hbm, idx_v, rows_v, sem):
        wid = lax.axis_index("s") * info.num_cores + lax.axis_index("c")
        base = wid * b_per_w
        pltpu.sync_copy(idx_hbm.at[pl.ds(base, b_per_w)], idx_v)
        pltpu.async_copy(table_hbm.at[idx_v], rows_v, sem).wait()
        pltpu.sync_copy(rows_v, out_hbm.at[pl.ds(base, b_per_w)])

    return k(table, idx)


def _stage3_body(num_ref, cat_ref, sp_ref, wnum_ref, ncst_ref, s_ref, a_ref,
                 w1c_ref, b1_ref, w2_ref, b2_ref, w3_ref, b3_ref, wfm_ref,
                 bfm_ref, out_ref):
    hp = lax.Precision.HIGHEST
    num = num_ref[...]
    cat = cat_ref[...]
    nsum = jnp.dot(num, wnum_ref[...], precision=hp) + ncst_ref[...]
    csum = jnp.dot(cat, s_ref[...], precision=hp)
    yfme = jnp.sum(nsum * csum, axis=1, keepdims=True)
    yfms = jnp.dot(num, wfm_ref[...], precision=hp) + sp_ref[...] + bfm_ref[...]
    h1 = jnp.maximum(
        jnp.dot(num, a_ref[...], precision=hp)
        + jnp.dot(cat, w1c_ref[...], precision=hp) + b1_ref[...], 0.0)
    h2 = jnp.maximum(jnp.dot(h1, w2_ref[...], precision=hp) + b2_ref[...], 0.0)
    yd = jnp.dot(h2, w3_ref[...], precision=hp) + b3_ref[...]
    out_ref[...] = jax.nn.sigmoid(yfme + yfms + yd)


def _stage3(num, catf, sparse, wnum, ncst, s, a, w1c, b1c, w2, b2, w3, b3,
            wfm13, bfm):
    return pl.pallas_call(
        _stage3_body,
        out_shape=jax.ShapeDtypeStruct((B, 1), jnp.float32),
    )(num, catf, sparse, wnum, ncst, s, a, w1c, b1c, w2, b2, w3, b3,
      wfm13, bfm)


def kernel(numeric_feats, categorical_feats, W_num, b_num, emb_tables,
           W_fm, b_fm, W1, b1, W2, b2, W3, b3):
    f32 = jnp.float32
    kk = jnp.arange(NCAT * V, dtype=jnp.int32)[:, None]
    cc = jnp.arange(NCOLS, dtype=jnp.int32)[None, :]
    d_hi = kk - cc * V
    d_lo = kk - (cc - NCAT) * V
    in_hi = (d_hi >= 0) & (d_hi < V)
    in_lo = (d_lo >= 0) & (d_lo < V) & (cc >= NCAT) & (cc < 2 * NCAT)
    w_b = jnp.broadcast_to(W_fm[NUM:, :1], (NCAT * V, NCOLS))
    m = jnp.where(in_hi, (d_hi >> 7).astype(f32),
                  jnp.where(in_lo, (d_lo & 127).astype(f32),
                            jnp.where(cc == 2 * NCAT, w_b, 0.0)))

    a = jnp.einsum('id,idm->im', W_num, W1[:NUM * D].reshape(NUM, D, -1))
    b1c = (b1 + b_num.reshape(-1) @ W1[:NUM * D])[None]
    ncst = b_num.sum(0)[None]
    s = jnp.tile(jnp.eye(D, dtype=f32), (NCAT, 1))

    idx2d, sparse = _stage1(categorical_feats, m)

    cat_rows = _sc_gather(emb_tables.reshape(NCAT * V, D), idx2d.reshape(-1))
    catf = cat_rows.reshape(B, NCAT * D)

    return _stage3(numeric_feats, catf, sparse, W_num, ncst, s, a,
                   W1[NUM * D:], b1c, W2, b2[None], W3, b3[None],
                   W_fm[:NUM], b_fm[None])
